# Initial kernel scaffold; baseline (speedup 1.0000x reference)
import jax
import jax.numpy as jnp
from jax.experimental import pallas as pl


def kernel(input_ids, word_table, pos_table):
    raise NotImplementedError("write your pallas kernel here")


# speedup vs baseline: 2.1035x; 2.1035x over previous
import functools

import jax
import jax.numpy as jnp
from jax import lax
from jax.experimental import pallas as pl
from jax.experimental.pallas import tpu as pltpu
from jax.experimental.pallas import tpu_sc as plsc

HIDDEN = 64
PAD_IDX = 0
SEQ = 200
LANES = 16

CHUNK = 800
NG = 10
GSLICE = CHUNK // NG
NCHUNKS = 1024
NW = 32
CPW = NCHUNKS // NW

_mesh = plsc.VectorSubcoreMesh(core_axis_name="c", subcore_axis_name="s")


@functools.partial(
    pl.kernel,
    out_type=jax.ShapeDtypeStruct((NCHUNKS, CHUNK, HIDDEN), jnp.float32),
    mesh=_mesh,
    compiler_params=pltpu.CompilerParams(use_tc_tiling_on_sc=False),
    scratch_types=[
        pltpu.VMEM((2, NG, GSLICE), jnp.int32),
        pltpu.VMEM((2, CHUNK, HIDDEN), jnp.float32),
        pltpu.VMEM((SEQ, HIDDEN), jnp.float32),
        pltpu.SemaphoreType.DMA,
        pltpu.SemaphoreType.DMA,
        pltpu.SemaphoreType.DMA,
        pltpu.SemaphoreType.DMA,
    ],
)
def _emb_lookup(ids_hbm, wt_hbm, pos_hbm, out_hbm, idx_v, rows_v, pos_v,
                gsem0, gsem1, osem0, osem1):
    gsems = (gsem0, gsem1)
    osems = (osem0, osem1)
    wid = lax.axis_index("s") * 2 + lax.axis_index("c")
    base = wid * CPW
    last = base + CPW - 1

    pltpu.sync_copy(pos_hbm.at[pl.ds(0, SEQ), :], pos_v)

    def fire_chunk(slot, c):
        pltpu.sync_copy(ids_hbm.at[c], idx_v.at[slot])
        for i in range(NG):
            pltpu.make_async_copy(
                wt_hbm.at[idx_v.at[slot, i]],
                rows_v.at[slot, pl.ds(i * GSLICE, GSLICE), :],
                gsems[slot],
            ).start()

    def drain_gather(slot):
        pltpu.make_async_copy(
            wt_hbm.at[pl.ds(0, CHUNK), :], rows_v.at[slot], gsems[slot]
        ).wait()

    def fire_out(slot, c):
        pltpu.make_async_copy(rows_v.at[slot], out_hbm.at[c], osems[slot]).start()

    def drain_out(slot):
        pltpu.make_async_copy(rows_v.at[slot], out_hbm.at[0], osems[slot]).wait()

    def compute(slot):
        groups_per_row = GSLICE // LANES

        def gbody(g, carry):
            i = g // groups_per_row
            col = lax.rem(g, groups_per_row) * LANES
            ivec = idx_v[slot, i, pl.ds(col, LANES)]
            mf = jnp.where(ivec == PAD_IDX, 0.0, 1.0).astype(jnp.float32)
            tbase = g * LANES
            jbase = lax.rem(tbase, SEQ)
            for k in range(LANES):
                t = tbase + k
                jj = jbase + k
                jj = jnp.where(jj >= SEQ, jj - SEQ, jj)
                mk = mf[k]
                for cc in range(HIDDEN // LANES):
                    sl = pl.ds(cc * LANES, LANES)
                    rows_v[slot, t, sl] = rows_v[slot, t, sl] * mk + pos_v[jj, sl]
            return carry

        lax.fori_loop(0, CHUNK // LANES, gbody, 0)

    def half(c, slot):
        other = 1 - slot

        @pl.when(c < last)
        def _prefetch():
            @pl.when(c > base)
            def _drain_prev_out():
                drain_out(other)

            fire_chunk(other, c + 1)

        drain_gather(slot)
        compute(slot)
        fire_out(slot, c)

    fire_chunk(0, base)

    def ibody(i, carry):
        c0 = base + 2 * i
        half(c0, 0)
        half(c0 + 1, 1)
        return carry

    lax.fori_loop(0, CPW // 2, ibody, 0)
    drain_out(0)
    drain_out(1)


def kernel(input_ids, word_table, pos_table):
    batch, seq = input_ids.shape
    ids3 = input_ids.reshape(NCHUNKS, NG, GSLICE)
    out = _emb_lookup(ids3, word_table, pos_table)
    return out.reshape(batch, seq, HIDDEN)
